# baseline (speedup 1.0000x reference)
import functools

import jax
import jax.numpy as jnp
from jax import lax
from jax.experimental import pallas as pl
from jax.experimental.pallas import tpu as pltpu
from jax.experimental.pallas import tpu_sc as plsc

EMBED = 64
HIDDEN = 256
CW = 128
NC = 2
NS = 16
NW = NC * NS
PBLK = 512


def _pack_body(x_ref, o_ref):
    t = x_ref[...].T.astype(jnp.bfloat16).astype(jnp.float32)
    bits = lax.bitcast_convert_type(t, jnp.int32)
    parts = []
    for k in range(4):
        bk = bits[k * 128:(k + 1) * 128]
        lo = lax.shift_right_logical(bk[:, :32], 16)
        hi = bk[:, 32:] & jnp.int32(-65536)
        parts.append(lo | hi)
    o_ref[...] = jnp.concatenate(parts, axis=1)


def _pack_table(tab):
    n = tab.shape[0]
    grid = pl.cdiv(n, PBLK)
    return pl.pallas_call(
        _pack_body,
        grid=(grid,),
        in_specs=[pl.BlockSpec((EMBED, PBLK), lambda b: (0, b))],
        out_specs=pl.BlockSpec((128, 2 * EMBED), lambda b: (b, 0)),
        out_shape=jax.ShapeDtypeStruct((grid * 128, 2 * EMBED), jnp.int32),
        compiler_params=pltpu.CompilerParams(
            dimension_semantics=("arbitrary",)),
    )(tab.T)


def _sc_gather(idx, packed):
    B = idx.shape[0]
    per = B // NW
    nch = per // CW
    mesh = plsc.VectorSubcoreMesh(core_axis_name="c", subcore_axis_name="s")

    @functools.partial(
        pl.kernel,
        out_type=jax.ShapeDtypeStruct((B, 2 * EMBED), jnp.int32),
        mesh=mesh,
        scratch_types=[
            pltpu.VMEM((nch, CW), jnp.int32),
            pltpu.VMEM((nch, CW, 2 * EMBED), jnp.int32),
            pltpu.SemaphoreType.DMA,
        ],
    )
    def gather(idx_hbm, tab_hbm, out_hbm, idxs, rows, sem):
        wid = lax.axis_index("s") * NC + lax.axis_index("c")
        base = wid * per
        for j in range(nch):
            pltpu.sync_copy(idx_hbm.at[pl.ds(base + j * CW, CW)], idxs.at[j])
        cps = []
        for j in range(nch):
            cps.append(pltpu.async_copy(
                tab_hbm.at[idxs.at[j]], rows.at[j], sem))
        for c in cps:
            c.wait()
        for j in range(nch):
            pltpu.sync_copy(rows.at[j],
                            out_hbm.at[pl.ds(base + j * CW, CW)])

    return gather(idx, packed)


def _mlp_body(gu_ref, gi_ref, su_ref, si_ref, w1t_ref, b1_ref, w2_ref,
              b2_ref, out_ref):
    def pick(g_ref, s_ref):
        g = g_ref[...]
        s = s_ref[...]

        def slot(k):
            xs = g[:, 32 * k:32 * k + 32]
            fl = lax.bitcast_convert_type(xs << 16, jnp.float32)
            fh = lax.bitcast_convert_type(xs & jnp.int32(-65536), jnp.float32)
            return jnp.concatenate([fl, fh], axis=1)

        r01 = jnp.where(s == 1, slot(1), slot(0))
        r23 = jnp.where(s == 3, slot(3), slot(2))
        return jnp.where(s >= 2, r23, r01)

    ue = pick(gu_ref, su_ref)
    ie = pick(gi_ref, si_ref)
    w1t = w1t_ref[...]
    h = (jnp.dot(ue, w1t[:EMBED], preferred_element_type=jnp.float32)
         + jnp.dot(ie, w1t[EMBED:], preferred_element_type=jnp.float32)
         + b1_ref[...])
    h = jnp.maximum(h, 0.0)
    out_ref[...] = jnp.sum(h * w2_ref[...], axis=1) + b2_ref[0]


def kernel(u, i, user_emb, item_emb, W1, b1, W2, b2):
    B = u.shape[0]
    u = u.astype(jnp.int32)
    i = i.astype(jnp.int32)

    up = _pack_table(user_emb)
    ip = _pack_table(item_emb)
    gu = _sc_gather((u // PBLK) * 128 + (u % 128), up)
    gi = _sc_gather((i // PBLK) * 128 + (i % 128), ip)

    su = ((u % PBLK) // 128).reshape(B, 1)
    si = ((i % PBLK) // 128).reshape(B, 1)

    W1T = W1.T
    b1r = b1.reshape(1, HIDDEN)

    BLK = 2048
    nblk = B // BLK
    out = pl.pallas_call(
        _mlp_body,
        grid=(nblk,),
        in_specs=[
            pl.BlockSpec((BLK, 2 * EMBED), lambda b: (b, 0)),
            pl.BlockSpec((BLK, 2 * EMBED), lambda b: (b, 0)),
            pl.BlockSpec((BLK, 1), lambda b: (b, 0)),
            pl.BlockSpec((BLK, 1), lambda b: (b, 0)),
            pl.BlockSpec((2 * EMBED, HIDDEN), lambda b: (0, 0)),
            pl.BlockSpec((1, HIDDEN), lambda b: (0, 0)),
            pl.BlockSpec((1, HIDDEN), lambda b: (0, 0)),
            pl.BlockSpec(memory_space=pltpu.SMEM),
        ],
        out_specs=pl.BlockSpec((BLK,), lambda b: (b,)),
        out_shape=jax.ShapeDtypeStruct((B,), jnp.float32),
        compiler_params=pltpu.CompilerParams(
            dimension_semantics=("arbitrary",)),
    )(gu, gi, su, si, W1T, b1r, W2, b2)
    return out


# speedup vs baseline: 3.5072x; 3.5072x over previous
import functools

import jax
import jax.numpy as jnp
from jax import lax
from jax.experimental import pallas as pl
from jax.experimental.pallas import tpu as pltpu
from jax.experimental.pallas import tpu_sc as plsc

EMBED = 64
HIDDEN = 256
CW = 128
NC = 2
NS = 16
NW = NC * NS
PBLK = 4096
EPB = PBLK // 4


def _pack_body(x_ref, o_ref):
    t = x_ref[...].T.astype(jnp.bfloat16).astype(jnp.float32)
    bits = lax.bitcast_convert_type(t, jnp.int32)
    parts = []
    for k in range(4):
        bk = bits[k * EPB:(k + 1) * EPB]
        lo = lax.shift_right_logical(bk[:, :32], 16)
        hi = bk[:, 32:] & jnp.int32(-65536)
        parts.append(lo | hi)
    o_ref[...] = jnp.concatenate(parts, axis=1)


def _pack_table(tab):
    n = tab.shape[0]
    grid = pl.cdiv(n, PBLK)
    return pl.pallas_call(
        _pack_body,
        grid=(grid,),
        in_specs=[pl.BlockSpec((EMBED, PBLK), lambda b: (0, b))],
        out_specs=pl.BlockSpec((EPB, 2 * EMBED), lambda b: (b, 0)),
        out_shape=jax.ShapeDtypeStruct((grid * EPB, 2 * EMBED), jnp.int32),
        compiler_params=pltpu.CompilerParams(
            dimension_semantics=("arbitrary",)),
    )(tab.T)


def _sc_gather(idx, packed):
    B = idx.shape[0]
    per = B // NW
    nch = per // CW
    mesh = plsc.VectorSubcoreMesh(core_axis_name="c", subcore_axis_name="s")

    @functools.partial(
        pl.kernel,
        out_type=jax.ShapeDtypeStruct((B, 2 * EMBED), jnp.int32),
        mesh=mesh,
        scratch_types=[
            pltpu.VMEM((nch, CW), jnp.int32),
            pltpu.VMEM((nch, CW, 2 * EMBED), jnp.int32),
            pltpu.SemaphoreType.DMA,
        ],
    )
    def gather(idx_hbm, tab_hbm, out_hbm, idxs, rows, sem):
        wid = lax.axis_index("s") * NC + lax.axis_index("c")
        base = wid * per
        for j in range(nch):
            pltpu.sync_copy(idx_hbm.at[pl.ds(base + j * CW, CW)], idxs.at[j])
        cps = []
        for j in range(nch):
            cps.append(pltpu.async_copy(
                tab_hbm.at[idxs.at[j]], rows.at[j], sem))
        for c in cps:
            c.wait()
        for j in range(nch):
            pltpu.sync_copy(rows.at[j],
                            out_hbm.at[pl.ds(base + j * CW, CW)])

    return gather(idx, packed)


def _mlp_body(gu_ref, gi_ref, su_ref, si_ref, w1t_ref, b1_ref, w2_ref,
              b2_ref, out_ref):
    def pick(g_ref, s_ref):
        g = g_ref[...]
        s = s_ref[...]

        def slot(k):
            xs = g[:, 32 * k:32 * k + 32]
            fl = lax.bitcast_convert_type(xs << 16, jnp.float32)
            fh = lax.bitcast_convert_type(xs & jnp.int32(-65536), jnp.float32)
            return jnp.concatenate([fl, fh], axis=1)

        r01 = jnp.where(s == 1, slot(1), slot(0))
        r23 = jnp.where(s == 3, slot(3), slot(2))
        return jnp.where(s >= 2, r23, r01)

    ue = pick(gu_ref, su_ref)
    ie = pick(gi_ref, si_ref)
    w1t = w1t_ref[...]
    h = (jnp.dot(ue, w1t[:EMBED], preferred_element_type=jnp.float32)
         + jnp.dot(ie, w1t[EMBED:], preferred_element_type=jnp.float32)
         + b1_ref[...])
    h = jnp.maximum(h, 0.0)
    out_ref[...] = jnp.sum(h * w2_ref[...], axis=1) + b2_ref[0]


def kernel(u, i, user_emb, item_emb, W1, b1, W2, b2):
    B = u.shape[0]
    u = u.astype(jnp.int32)
    i = i.astype(jnp.int32)

    up = _pack_table(user_emb)
    ip = _pack_table(item_emb)
    gu = _sc_gather((u // PBLK) * EPB + (u % EPB), up)
    gi = _sc_gather((i // PBLK) * EPB + (i % EPB), ip)

    su = ((u % PBLK) // EPB).reshape(B, 1)
    si = ((i % PBLK) // EPB).reshape(B, 1)

    W1T = W1.T
    b1r = b1.reshape(1, HIDDEN)

    BLK = 2048
    nblk = B // BLK
    out = pl.pallas_call(
        _mlp_body,
        grid=(nblk,),
        in_specs=[
            pl.BlockSpec((BLK, 2 * EMBED), lambda b: (b, 0)),
            pl.BlockSpec((BLK, 2 * EMBED), lambda b: (b, 0)),
            pl.BlockSpec((BLK, 1), lambda b: (b, 0)),
            pl.BlockSpec((BLK, 1), lambda b: (b, 0)),
            pl.BlockSpec((2 * EMBED, HIDDEN), lambda b: (0, 0)),
            pl.BlockSpec((1, HIDDEN), lambda b: (0, 0)),
            pl.BlockSpec((1, HIDDEN), lambda b: (0, 0)),
            pl.BlockSpec(memory_space=pltpu.SMEM),
        ],
        out_specs=pl.BlockSpec((BLK,), lambda b: (b,)),
        out_shape=jax.ShapeDtypeStruct((B,), jnp.float32),
        compiler_params=pltpu.CompilerParams(
            dimension_semantics=("arbitrary",)),
    )(gu, gi, su, si, W1T, b1r, W2, b2)
    return out
